# Initial kernel scaffold; baseline (speedup 1.0000x reference)
import jax
import jax.numpy as jnp
from jax import lax
from jax.experimental import pallas as pl
from jax.experimental.pallas import tpu as pltpu
from jax.experimental.pallas import tpu_sc as plsc

N = 10000
E = 320000
D = 128

NC = 2
NS = 16
NW = NC * NS

CHUNK = 128
E_PER_W = -(-E // (NW * CHUNK)) * CHUNK
NCH = E_PER_W // CHUNK
E_PAD = E_PER_W * NW
ROWS_PER_TILE = N // NS


def _sc_body(x_hbm, row_hbm, col_hbm, zero_hbm, out_hbm, row_v, col_v, gbuf, sem, agg_sh):
    cid = lax.axis_index("c")
    sid = lax.axis_index("s")
    wid = sid * NC + cid

    pltpu.sync_copy(
        zero_hbm.at[pl.ds(sid * ROWS_PER_TILE, ROWS_PER_TILE)],
        agg_sh.at[pl.ds(sid * ROWS_PER_TILE, ROWS_PER_TILE)],
    )
    plsc.subcore_barrier()

    pltpu.sync_copy(row_hbm.at[wid], row_v)
    pltpu.sync_copy(col_hbm.at[wid], col_v)

    def step(i, carry):
        pltpu.async_copy(x_hbm.at[col_v.at[i]], gbuf, sem).wait()
        pltpu.sync_copy(gbuf, agg_sh.at[row_v.at[i]], add=True)
        return carry

    lax.fori_loop(0, NCH, step, 0)
    plsc.subcore_barrier()

    pltpu.sync_copy(
        agg_sh.at[pl.ds(sid * ROWS_PER_TILE, ROWS_PER_TILE)],
        out_hbm.at[cid, pl.ds(sid * ROWS_PER_TILE, ROWS_PER_TILE)],
    )


@jax.jit
def _sc_aggregate(x_pad, row3, col3, zeros):
    mesh = plsc.VectorSubcoreMesh(core_axis_name="c", subcore_axis_name="s")
    return pl.kernel(
        _sc_body,
        out_type=jax.ShapeDtypeStruct((NC, N, D), jnp.float32),
        mesh=mesh,
        scratch_types=[
            pltpu.VMEM((NCH, CHUNK), jnp.int32),
            pltpu.VMEM((NCH, CHUNK), jnp.int32),
            pltpu.VMEM((CHUNK, D), jnp.float32),
            pltpu.SemaphoreType.DMA,
            pltpu.VMEM_SHARED((N, D), jnp.float32),
        ],
    )(x_pad, row3, col3, zeros)


def _tc_body(x_ref, p_ref, w1_ref, b1_ref, w2_ref, b2_ref, g_ref, bt_ref, o_ref):
    h = x_ref[...] + p_ref[0] + p_ref[1]
    h = lax.dot_general(h, w1_ref[...], (((1,), (1,)), ((), ())),
                        preferred_element_type=jnp.float32) + b1_ref[...]
    h = jnp.maximum(h, 0.0)
    h = lax.dot_general(h, w2_ref[...], (((1,), (1,)), ((), ())),
                        preferred_element_type=jnp.float32) + b2_ref[...]
    mean = jnp.mean(h, axis=0)
    var = jnp.mean(h * h, axis=0) - mean * mean
    o_ref[...] = (h - mean) * lax.rsqrt(var + 1e-5) * g_ref[...] + bt_ref[...]


@jax.jit
def _tc_mlp_bn(x, partials, W1, b1, W2, b2, gamma, beta):
    return pl.pallas_call(
        _tc_body,
        out_shape=jax.ShapeDtypeStruct((N, D), jnp.float32),
    )(x, partials, W1, b1, W2, b2, gamma, beta)


def kernel(x, edge_index, W1, b1, W2, b2, gamma, beta):
    row = edge_index[0].astype(jnp.int32)
    col = edge_index[1].astype(jnp.int32)
    pad = E_PAD - E
    row_p = jnp.concatenate([row, jnp.zeros((pad,), jnp.int32)])
    col_p = jnp.concatenate([col, jnp.full((pad,), N, jnp.int32)])
    x_pad = jnp.concatenate([x, jnp.zeros((1, D), jnp.float32)])
    row3 = row_p.reshape(NW, NCH, CHUNK)
    col3 = col_p.reshape(NW, NCH, CHUNK)
    zeros = jnp.zeros((N, D), jnp.float32)
    partials = _sc_aggregate(x_pad, row3, col3, zeros)
    return _tc_mlp_bn(x, partials, W1, b1, W2, b2, gamma, beta)


# speedup vs baseline: 3.9862x; 3.9862x over previous
import jax
import jax.numpy as jnp
from jax import lax
from jax.experimental import pallas as pl
from jax.experimental.pallas import tpu as pltpu
from jax.experimental.pallas import tpu_sc as plsc

N = 10000
E = 320000
D = 128

NC = 2
NS = 16
NW = NC * NS

CHUNK = 128
E_PER_W = -(-E // (NW * CHUNK)) * CHUNK
NCH = E_PER_W // CHUNK
E_PAD = E_PER_W * NW
N_PAD = 10112
ROWS_PER_TILE = N_PAD // NS


def _sc_body(x_hbm, row_hbm, col_hbm, zero_hbm, out_hbm, row_v, col_v, gbuf, sem, agg_sh):
    cid = lax.axis_index("c")
    sid = lax.axis_index("s")
    wid = sid * NC + cid

    pltpu.sync_copy(
        zero_hbm.at[pl.ds(sid * ROWS_PER_TILE, ROWS_PER_TILE)],
        agg_sh.at[pl.ds(sid * ROWS_PER_TILE, ROWS_PER_TILE)],
    )
    plsc.subcore_barrier()

    pltpu.sync_copy(row_hbm.at[wid], row_v)
    pltpu.sync_copy(col_hbm.at[wid], col_v)

    def step(i, carry):
        pltpu.async_copy(x_hbm.at[col_v.at[i]], gbuf, sem).wait()
        pltpu.sync_copy(gbuf, agg_sh.at[row_v.at[i]], add=True)
        return carry

    lax.fori_loop(0, NCH, step, 0)
    plsc.subcore_barrier()

    pltpu.sync_copy(
        agg_sh.at[pl.ds(sid * ROWS_PER_TILE, ROWS_PER_TILE)],
        out_hbm.at[cid, pl.ds(sid * ROWS_PER_TILE, ROWS_PER_TILE)],
    )


@jax.jit
def _sc_aggregate(x_pad, row3, col3, zeros):
    mesh = plsc.VectorSubcoreMesh(core_axis_name="c", subcore_axis_name="s")
    return pl.kernel(
        _sc_body,
        out_type=jax.ShapeDtypeStruct((NC, N_PAD, D), jnp.float32),
        mesh=mesh,
        scratch_types=[
            pltpu.VMEM((NCH, CHUNK), jnp.int32),
            pltpu.VMEM((NCH, CHUNK), jnp.int32),
            pltpu.VMEM((CHUNK, D), jnp.float32),
            pltpu.SemaphoreType.DMA,
            pltpu.VMEM_SHARED((N_PAD, D), jnp.float32),
        ],
    )(x_pad, row3, col3, zeros)


def _tc_body(x_ref, p_ref, w1_ref, b1_ref, w2_ref, b2_ref, g_ref, bt_ref, o_ref):
    h = x_ref[...] + p_ref[0, :N, :] + p_ref[1, :N, :]
    h = lax.dot_general(h, w1_ref[...], (((1,), (1,)), ((), ())),
                        preferred_element_type=jnp.float32) + b1_ref[...]
    h = jnp.maximum(h, 0.0)
    h = lax.dot_general(h, w2_ref[...], (((1,), (1,)), ((), ())),
                        preferred_element_type=jnp.float32) + b2_ref[...]
    mean = jnp.mean(h, axis=0)
    var = jnp.mean(h * h, axis=0) - mean * mean
    o_ref[...] = (h - mean) * lax.rsqrt(var + 1e-5) * g_ref[...] + bt_ref[...]


@jax.jit
def _tc_mlp_bn(x, partials, W1, b1, W2, b2, gamma, beta):
    return pl.pallas_call(
        _tc_body,
        out_shape=jax.ShapeDtypeStruct((N, D), jnp.float32),
    )(x, partials, W1, b1, W2, b2, gamma, beta)


def kernel(x, edge_index, W1, b1, W2, b2, gamma, beta):
    row = edge_index[0].astype(jnp.int32)
    col = edge_index[1].astype(jnp.int32)
    pad = E_PAD - E
    row_p = jnp.concatenate([row, jnp.zeros((pad,), jnp.int32)])
    col_p = jnp.concatenate([col, jnp.full((pad,), N, jnp.int32)])
    x_pad = jnp.concatenate([x, jnp.zeros((1, D), jnp.float32)])
    row3 = row_p.reshape(NW, NCH, CHUNK)
    col3 = col_p.reshape(NW, NCH, CHUNK)
    zeros = jnp.zeros((N_PAD, D), jnp.float32)
    partials = _sc_aggregate(x_pad, row3, col3, zeros)
    return _tc_mlp_bn(x, partials, W1, b1, W2, b2, gamma, beta)
